# Initial kernel scaffold; baseline (speedup 1.0000x reference)
import jax
import jax.numpy as jnp
from jax.experimental import pallas as pl


def kernel(word_ids, topic_ids, ww_src, ww_dst, wt_src, wt_dst, tt_src, tt_dst, wd_src, wd_dst, td_src, td_dst, doc_gid, y_data, W_word, topic_embeds, adapt_W, adapt_b, layers, out_W, out_b):
    raise NotImplementedError("write your pallas kernel here")


# speedup vs baseline: 2.5394x; 2.5394x over previous
import functools
import jax
import jax.numpy as jnp
from jax import lax
from jax.experimental import pallas as pl
from jax.experimental.pallas import tpu as pltpu
from jax.experimental.pallas import tpu_sc as plsc

NC, NS = 2, 16
NW = NC * NS
CH = 128

NWORD, NWORD_P = 16000, 16128
NTOPIC, NTOPIC_P = 3200, 3328
NDOC, NDOC_P = 64, 128

WW_HALF = 8064
WW_ACC = 8192

WW_E, WW_CPT = 258048, 126
WT_E, WT_CPT = 65536, 32
TT_E, TT_CPT = 8192, 4
WD_E, WD_CPT = 131072, 32
TD_E, TD_CPT = 8192, 2

F32 = jnp.float32
_mesh = functools.partial(plsc.VectorSubcoreMesh,
                          core_axis_name="c", subcore_axis_name="s")


def _pad_edges(src, dst, sac_row, mult):
  e = src.shape[0]
  ep = ((e + mult - 1) // mult) * mult
  src = src.astype(jnp.int32)
  dst = dst.astype(jnp.int32)
  if ep != e:
    src = jnp.concatenate([src, jnp.zeros((ep - e,), jnp.int32)])
    dst = jnp.concatenate([dst, jnp.full((ep - e,), sac_row, jnp.int32)])
  return src, dst


def _remap(didx, didx2, lo, n):
  for k in range(CH // 16):
    d = didx[pl.ds(k * 16, 16)] - lo
    ok = (d >= 0) & (d < n)
    didx2[pl.ds(k * 16, 16)] = jnp.where(ok, d, n)


def _prep_body(w_word, word_ids, t_emb, topic_ids, ww_dst, wt_dst, tt_dst,
               z128, ones128,
               word_rows, topic_rows, cnt_ww, cnt_wt, cnt_tt,
               cww, ct, idx_v, didx2, rows_v, ones_v, sem):
  c = lax.axis_index("c")
  s = lax.axis_index("s")
  w = s * NC + c
  lo = c * WW_HALF
  pltpu.sync_copy(z128.at[pl.ds(0, 512)], cww.at[pl.ds(s * 512, 512)])
  pltpu.sync_copy(z128.at[pl.ds(0, 208)], ct.at[pl.ds(s * 208, 208)])
  pltpu.sync_copy(ones128, ones_v)

  def wg(i, _):
    g = w + i * NW

    @pl.when(g < 125)
    def _():
      pltpu.sync_copy(word_ids.at[pl.ds(g * CH, CH)], idx_v)
      pltpu.async_copy(w_word.at[idx_v], rows_v, sem).wait()
      pltpu.sync_copy(rows_v, word_rows.at[pl.ds(g * CH, CH)])
    return 0

  lax.fori_loop(0, 4, wg, 0)

  @pl.when(w < 25)
  def _():
    pltpu.sync_copy(topic_ids.at[pl.ds(w * CH, CH)], idx_v)
    pltpu.async_copy(t_emb.at[idx_v], rows_v, sem).wait()
    pltpu.sync_copy(rows_v, topic_rows.at[pl.ds(w * CH, CH)])

  plsc.subcore_barrier()

  def cg(i, _):
    off = (s * WW_CPT + i) * CH
    pltpu.sync_copy(ww_dst.at[pl.ds(off, CH)], idx_v)
    _remap(idx_v, didx2, lo, WW_HALF)
    pltpu.sync_copy(ones_v, cww.at[didx2], add=True)
    return 0

  lax.fori_loop(0, WW_CPT, cg, 0)

  @pl.when(c == 0)
  def _():
    def cwt(i, _):
      off = (s * WT_CPT + i) * CH
      pltpu.sync_copy(wt_dst.at[pl.ds(off, CH)], idx_v)
      pltpu.sync_copy(ones_v, ct.at[idx_v], add=True)
      return 0
    lax.fori_loop(0, WT_CPT, cwt, 0)

  @pl.when(c == 1)
  def _():
    def ctt(i, _):
      off = (s * TT_CPT + i) * CH
      pltpu.sync_copy(tt_dst.at[pl.ds(off, CH)], idx_v)
      pltpu.sync_copy(ones_v, ct.at[idx_v], add=True)
      return 0
    lax.fori_loop(0, TT_CPT, ctt, 0)

  plsc.subcore_barrier()
  pltpu.sync_copy(cww.at[pl.ds(s * 504, 504)],
                  cnt_ww.at[pl.ds(lo + s * 504, 504)])

  @pl.when(c == 0)
  def _():
    pltpu.sync_copy(ct.at[pl.ds(s * 208, 208)],
                    cnt_wt.at[pl.ds(s * 208, 208)])

  @pl.when(c == 1)
  def _():
    pltpu.sync_copy(ct.at[pl.ds(s * 208, 208)],
                    cnt_tt.at[pl.ds(s * 208, 208)])


def _l1a_body(feat_w, src, dst, z128,
              s_ww,
              acc, sidx, didx, didx2, rows_v, sem):
  c = lax.axis_index("c")
  s = lax.axis_index("s")
  lo = c * WW_HALF
  pltpu.sync_copy(z128.at[pl.ds(0, 512)], acc.at[pl.ds(s * 512, 512)])
  plsc.subcore_barrier()

  def body(i, _):
    off = (s * WW_CPT + i) * CH
    pltpu.sync_copy(src.at[pl.ds(off, CH)], sidx)
    pltpu.sync_copy(dst.at[pl.ds(off, CH)], didx)
    _remap(didx, didx2, lo, WW_HALF)
    pltpu.async_copy(feat_w.at[sidx], rows_v, sem).wait()
    pltpu.sync_copy(rows_v, acc.at[didx2], add=True)
    return 0

  lax.fori_loop(0, WW_CPT, body, 0)
  plsc.subcore_barrier()
  pltpu.sync_copy(acc.at[pl.ds(s * 504, 504)],
                  s_ww.at[pl.ds(lo + s * 504, 504)])


def _l1b_body(feat_w, feat_t, wt_src, wt_dst, tt_src, tt_dst, z128,
              s_wt, s_tt,
              acc, sidx, didx, rows_v, sem):
  c = lax.axis_index("c")
  s = lax.axis_index("s")
  pltpu.sync_copy(z128.at[pl.ds(0, 208)], acc.at[pl.ds(s * 208, 208)])
  plsc.subcore_barrier()

  def agg(feat, esrc, edst, n):
    def body(i, _):
      off = (s * n + i) * CH
      pltpu.sync_copy(esrc.at[pl.ds(off, CH)], sidx)
      pltpu.sync_copy(edst.at[pl.ds(off, CH)], didx)
      pltpu.async_copy(feat.at[sidx], rows_v, sem).wait()
      pltpu.sync_copy(rows_v, acc.at[didx], add=True)
      return 0
    lax.fori_loop(0, n, body, 0)

  @pl.when(c == 0)
  def _():
    agg(feat_w, wt_src, wt_dst, WT_CPT)

  @pl.when(c == 1)
  def _():
    agg(feat_t, tt_src, tt_dst, TT_CPT)

  plsc.subcore_barrier()

  @pl.when(c == 0)
  def _():
    pltpu.sync_copy(acc.at[pl.ds(s * 208, 208)],
                    s_wt.at[pl.ds(s * 208, 208)])

  @pl.when(c == 1)
  def _():
    pltpu.sync_copy(acc.at[pl.ds(s * 208, 208)],
                    s_tt.at[pl.ds(s * 208, 208)])


def _l2_body(feat_w, feat_t, wd_src, wd_dst, td_src, td_dst, z128, ones128,
             s_wd, s_td, cnt_wd, cnt_td,
             a_wd, a_td, cw, ct, sidx, didx, rows_v, ones_v, sem):
  c = lax.axis_index("c")
  s = lax.axis_index("s")
  for a in (a_wd, a_td, cw, ct):
    pltpu.sync_copy(z128.at[pl.ds(0, 8)], a.at[pl.ds(s * 8, 8)])
  pltpu.sync_copy(ones128, ones_v)
  plsc.subcore_barrier()

  def agg(feat, esrc, edst, acc, cacc, n):
    def body(i, _):
      off = ((c * NS + s) * n + i) * CH
      pltpu.sync_copy(esrc.at[pl.ds(off, CH)], sidx)
      pltpu.sync_copy(edst.at[pl.ds(off, CH)], didx)
      pltpu.async_copy(feat.at[sidx], rows_v, sem).wait()
      pltpu.sync_copy(rows_v, acc.at[didx], add=True)
      pltpu.sync_copy(ones_v, cacc.at[didx], add=True)
      return 0
    lax.fori_loop(0, n, body, 0)

  agg(feat_w, wd_src, wd_dst, a_wd, cw, WD_CPT)
  agg(feat_t, td_src, td_dst, a_td, ct, TD_CPT)
  plsc.subcore_barrier()
  for a, o in ((a_wd, s_wd), (a_td, s_td), (cw, cnt_wd), (ct, cnt_td)):
    pltpu.sync_copy(a.at[pl.ds(s * 8, 8)], o.at[c, pl.ds(s * 8, 8)])


def _adapt_body(rows_ref, w_ref, b_ref, o_ref):
  o_ref[...] = jnp.dot(rows_ref[...], w_ref[...],
                       preferred_element_type=F32) + b_ref[...]


def _word1_body(s_ref, c_ref, w_ref, b_ref, o_ref):
  cnt = c_ref[:, 0:1]
  num = jnp.dot(s_ref[...], w_ref[...],
                preferred_element_type=F32) + cnt * b_ref[...]
  o_ref[...] = jnp.maximum(num / jnp.maximum(cnt, 1.0), 0.0)


def _mean_block(sagg, cnt, w_ref, b_ref):
  num = jnp.dot(sagg, w_ref[...], preferred_element_type=F32) + cnt * b_ref[...]
  return num / jnp.maximum(cnt, 1.0)


def _topic1_body(swt_ref, cwt_ref, wwt_ref, bwt_ref,
                 stt_ref, ctt_ref, wtt_ref, btt_ref, o_ref):
  m = (_mean_block(swt_ref[...], cwt_ref[:, 0:1], wwt_ref, bwt_ref) +
       _mean_block(stt_ref[...], ctt_ref[:, 0:1], wtt_ref, btt_ref))
  o_ref[...] = jnp.maximum(m, 0.0)


def _head_body(swd_ref, cwd_ref, wwd_ref, bwd_ref,
               std_ref, ctd_ref, wtd_ref, btd_ref,
               ow_ref, ob_ref, y_ref, loss_ref, yp_ref):
  m = (_mean_block(swd_ref[0] + swd_ref[1],
                   cwd_ref[0][:, 0:1] + cwd_ref[1][:, 0:1],
                   wwd_ref, bwd_ref) +
       _mean_block(std_ref[0] + std_ref[1],
                   ctd_ref[0][:, 0:1] + ctd_ref[1][:, 0:1],
                   wtd_ref, btd_ref))
  feat = jnp.maximum(m[0:NDOC], 0.0)
  z = jnp.dot(feat, ow_ref[...], preferred_element_type=F32) + ob_ref[...]
  y = y_ref[...]
  loss = jnp.mean(jnp.maximum(z, 0.0) - z * y +
                  jnp.log1p(jnp.exp(-jnp.abs(z))))
  loss_ref[...] = loss[None, None]
  yp_ref[...] = jax.nn.sigmoid(z)


def kernel(word_ids, topic_ids, ww_src, ww_dst, wt_src, wt_dst, tt_src,
           tt_dst, wd_src, wd_dst, td_src, td_dst, doc_gid, y_data,
           W_word, topic_embeds, adapt_W, adapt_b, layers, out_W, out_b):
  mult = NW * CH
  ww_s, ww_d = _pad_edges(ww_src, ww_dst, NWORD, mult)
  wt_s, wt_d = _pad_edges(wt_src, wt_dst, NTOPIC, mult)
  tt_s, tt_d = _pad_edges(tt_src, tt_dst, NTOPIC, mult)
  wd_s, wd_d = _pad_edges(wd_src, wd_dst, NDOC, mult)
  td_s, td_d = _pad_edges(td_src, td_dst, NDOC, mult)
  word_ids = word_ids.astype(jnp.int32)
  topic_ids = topic_ids.astype(jnp.int32)

  z128 = jnp.zeros((512, 128), F32)
  ones128 = jnp.ones((CH, 128), F32)

  sds = jax.ShapeDtypeStruct

  prep = pl.kernel(
      _prep_body,
      out_type=(sds((NWORD, 128), F32), sds((NTOPIC, 128), F32),
                sds((NWORD_P, 128), F32), sds((NTOPIC_P, 128), F32),
                sds((NTOPIC_P, 128), F32)),
      mesh=_mesh(),
      scratch_types=(pltpu.VMEM_SHARED((WW_ACC, 128), F32),
                     pltpu.VMEM_SHARED((NTOPIC_P, 128), F32),
                     pltpu.VMEM((CH,), jnp.int32),
                     pltpu.VMEM((CH,), jnp.int32),
                     pltpu.VMEM((CH, 128), F32),
                     pltpu.VMEM((CH, 128), F32),
                     pltpu.SemaphoreType.DMA),
  )
  word_rows, feat_t0, cnt_ww, cnt_wt, cnt_tt = prep(
      W_word, word_ids, topic_embeds, topic_ids, ww_d, wt_d, tt_d,
      z128, ones128)

  feat_w0 = pl.pallas_call(
      _adapt_body,
      grid=(10,),
      in_specs=[pl.BlockSpec((1600, 128), lambda i: (i, 0)),
                pl.BlockSpec((128, 128), lambda i: (0, 0)),
                pl.BlockSpec((1, 128), lambda i: (0, 0))],
      out_specs=pl.BlockSpec((1600, 128), lambda i: (i, 0)),
      out_shape=sds((NWORD, 128), F32),
  )(word_rows, adapt_W, adapt_b.reshape(1, 128))

  l1a = pl.kernel(
      _l1a_body,
      out_type=sds((NWORD_P, 128), F32),
      mesh=_mesh(),
      scratch_types=(pltpu.VMEM_SHARED((WW_ACC, 128), F32),
                     pltpu.VMEM((CH,), jnp.int32),
                     pltpu.VMEM((CH,), jnp.int32),
                     pltpu.VMEM((CH,), jnp.int32),
                     pltpu.VMEM((CH, 128), F32),
                     pltpu.SemaphoreType.DMA),
  )
  s_ww = l1a(feat_w0, ww_s, ww_d, z128)

  l1b = pl.kernel(
      _l1b_body,
      out_type=(sds((NTOPIC_P, 128), F32), sds((NTOPIC_P, 128), F32)),
      mesh=_mesh(),
      scratch_types=(pltpu.VMEM_SHARED((NTOPIC_P, 128), F32),
                     pltpu.VMEM((CH,), jnp.int32),
                     pltpu.VMEM((CH,), jnp.int32),
                     pltpu.VMEM((CH, 128), F32),
                     pltpu.SemaphoreType.DMA),
  )
  s_wt, s_tt = l1b(feat_w0, feat_t0, wt_s, wt_d, tt_s, tt_d, z128)

  lp1, lp2 = layers[0], layers[1]

  feat_w1 = pl.pallas_call(
      _word1_body,
      grid=(16,),
      in_specs=[pl.BlockSpec((1008, 128), lambda i: (i, 0)),
                pl.BlockSpec((1008, 128), lambda i: (i, 0)),
                pl.BlockSpec((128, 128), lambda i: (0, 0)),
                pl.BlockSpec((1, 128), lambda i: (0, 0))],
      out_specs=pl.BlockSpec((1008, 128), lambda i: (i, 0)),
      out_shape=sds((NWORD_P, 128), F32),
  )(s_ww, cnt_ww, lp1['ww']['W'], lp1['ww']['b'].reshape(1, 128))

  feat_t1 = pl.pallas_call(
      _topic1_body,
      in_specs=[pl.BlockSpec((NTOPIC_P, 128), lambda: (0, 0)),
                pl.BlockSpec((NTOPIC_P, 128), lambda: (0, 0)),
                pl.BlockSpec((128, 128), lambda: (0, 0)),
                pl.BlockSpec((1, 128), lambda: (0, 0)),
                pl.BlockSpec((NTOPIC_P, 128), lambda: (0, 0)),
                pl.BlockSpec((NTOPIC_P, 128), lambda: (0, 0)),
                pl.BlockSpec((128, 128), lambda: (0, 0)),
                pl.BlockSpec((1, 128), lambda: (0, 0))],
      out_specs=pl.BlockSpec((NTOPIC_P, 128), lambda: (0, 0)),
      out_shape=sds((NTOPIC_P, 128), F32),
  )(s_wt, cnt_wt, lp1['wt']['W'], lp1['wt']['b'].reshape(1, 128),
    s_tt, cnt_tt, lp1['tt']['W'], lp1['tt']['b'].reshape(1, 128))

  l2 = pl.kernel(
      _l2_body,
      out_type=(sds((NC, NDOC_P, 128), F32), sds((NC, NDOC_P, 128), F32),
                sds((NC, NDOC_P, 128), F32), sds((NC, NDOC_P, 128), F32)),
      mesh=_mesh(),
      scratch_types=(pltpu.VMEM_SHARED((NDOC_P, 128), F32),
                     pltpu.VMEM_SHARED((NDOC_P, 128), F32),
                     pltpu.VMEM_SHARED((NDOC_P, 128), F32),
                     pltpu.VMEM_SHARED((NDOC_P, 128), F32),
                     pltpu.VMEM((CH,), jnp.int32),
                     pltpu.VMEM((CH,), jnp.int32),
                     pltpu.VMEM((CH, 128), F32),
                     pltpu.VMEM((CH, 128), F32),
                     pltpu.SemaphoreType.DMA),
  )
  s_wd, s_td, cnt_wd, cnt_td = l2(feat_w1, feat_t1, wd_s, wd_d, td_s, td_d,
                                  z128, ones128)

  loss2d, yp2d = pl.pallas_call(
      _head_body,
      in_specs=[pl.BlockSpec((NC, NDOC_P, 128), lambda: (0, 0, 0)),
                pl.BlockSpec((NC, NDOC_P, 128), lambda: (0, 0, 0)),
                pl.BlockSpec((128, 128), lambda: (0, 0)),
                pl.BlockSpec((1, 128), lambda: (0, 0)),
                pl.BlockSpec((NC, NDOC_P, 128), lambda: (0, 0, 0)),
                pl.BlockSpec((NC, NDOC_P, 128), lambda: (0, 0, 0)),
                pl.BlockSpec((128, 128), lambda: (0, 0)),
                pl.BlockSpec((1, 128), lambda: (0, 0)),
                pl.BlockSpec((128, 1), lambda: (0, 0)),
                pl.BlockSpec((1, 1), lambda: (0, 0)),
                pl.BlockSpec((NDOC, 1), lambda: (0, 0))],
      out_specs=[pl.BlockSpec((1, 1), lambda: (0, 0)),
                 pl.BlockSpec((NDOC, 1), lambda: (0, 0))],
      out_shape=(sds((1, 1), F32), sds((NDOC, 1), F32)),
  )(s_wd, cnt_wd, lp2['wd']['W'], lp2['wd']['b'].reshape(1, 128),
    s_td, cnt_td, lp2['td']['W'], lp2['td']['b'].reshape(1, 128),
    out_W, out_b.reshape(1, 1), y_data.reshape(NDOC, 1))

  return (loss2d.reshape(()), yp2d.reshape(NDOC))


# speedup vs baseline: 2.9358x; 1.1561x over previous
import functools
import jax
import jax.numpy as jnp
from jax import lax
from jax.experimental import pallas as pl
from jax.experimental.pallas import tpu as pltpu
from jax.experimental.pallas import tpu_sc as plsc

NC, NS = 2, 16
NW = NC * NS
CH = 128

NWORD, NWORD_P = 16000, 16128
NTOPIC, NTOPIC_P = 3200, 3328
NDOC, NDOC_P = 64, 128

WW_HALF = 8064
WW_ACC = 8192

WW_E, WW_CPT = 258048, 126
WT_E, WT_CPT = 65536, 32
TT_E, TT_CPT = 8192, 4
WD_E, WD_CPT = 131072, 32
TD_E, TD_CPT = 8192, 2

F32 = jnp.float32
_mesh = functools.partial(plsc.VectorSubcoreMesh,
                          core_axis_name="c", subcore_axis_name="s")


def _pad_edges(src, dst, sac_row, mult):
  e = src.shape[0]
  ep = ((e + mult - 1) // mult) * mult
  src = src.astype(jnp.int32)
  dst = dst.astype(jnp.int32)
  if ep != e:
    src = jnp.concatenate([src, jnp.zeros((ep - e,), jnp.int32)])
    dst = jnp.concatenate([dst, jnp.full((ep - e,), sac_row, jnp.int32)])
  return src, dst


def _remap(didx_big, b, didx2, lo, n):
  for k in range(CH // 16):
    d = didx_big[pl.ds(b * CH + k * 16, 16)]
    if lo is not None:
      d = d - lo
      d = jnp.where((d >= 0) & (d < n), d, n)
    didx2[b, pl.ds(k * 16, 16)] = d


def _agg_groups(esrc, edst, feat, acc, cacc, sidx_big, didx_big, didx2,
                rows, ones_v, gsem, ssem, base_chunk, n_chunks, K,
                lo=None, nrange=None):
  ng = n_chunks // K

  def grp(g, _):
    off0 = (base_chunk + g * K) * CH
    pltpu.sync_copy(edst.at[pl.ds(off0, K * CH)], didx_big.at[pl.ds(0, K * CH)])
    if feat is not None:
      pltpu.sync_copy(esrc.at[pl.ds(off0, K * CH)],
                      sidx_big.at[pl.ds(0, K * CH)])
      for b in range(K):
        pltpu.async_copy(feat.at[sidx_big.at[pl.ds(b * CH, CH)]],
                         rows.at[b], gsem)
    for b in range(K):
      _remap(didx_big, b, didx2, lo, nrange)
    if cacc is not None:
      for b in range(K):
        pltpu.async_copy(ones_v, cacc.at[didx2.at[b]], ssem, add=True)
    if feat is not None:
      for b in range(K):
        pltpu.make_async_copy(feat.at[sidx_big.at[pl.ds(b * CH, CH)]],
                              rows.at[b], gsem).wait()
      for b in range(K):
        pltpu.async_copy(rows.at[b], acc.at[didx2.at[b]], ssem, add=True)
      for b in range(K):
        pltpu.make_async_copy(rows.at[b], acc.at[didx2.at[b]], ssem).wait()
    if cacc is not None:
      for b in range(K):
        pltpu.make_async_copy(ones_v, cacc.at[didx2.at[b]], ssem).wait()
    return 0

  lax.fori_loop(0, ng, grp, 0)


def _prep_body(w_word, word_ids, t_emb, topic_ids, ww_dst, wt_dst, tt_dst,
               z128, ones128,
               word_rows, topic_rows, cnt_ww, cnt_wt, cnt_tt,
               cww, ct, idx_v, didx_big, didx2, rows_v, ones_v, gsem, ssem):
  c = lax.axis_index("c")
  s = lax.axis_index("s")
  w = s * NC + c
  lo = c * WW_HALF
  pltpu.sync_copy(z128.at[pl.ds(0, 512)], cww.at[pl.ds(s * 512, 512)])
  pltpu.sync_copy(z128.at[pl.ds(0, 208)], ct.at[pl.ds(s * 208, 208)])
  pltpu.sync_copy(ones128, ones_v)

  def wg(i, _):
    g = w + i * NW

    @pl.when(g < 125)
    def _():
      pltpu.sync_copy(word_ids.at[pl.ds(g * CH, CH)], idx_v)
      pltpu.async_copy(w_word.at[idx_v], rows_v, gsem).wait()
      pltpu.sync_copy(rows_v, word_rows.at[pl.ds(g * CH, CH)])
    return 0

  lax.fori_loop(0, 4, wg, 0)

  @pl.when(w < 25)
  def _():
    pltpu.sync_copy(topic_ids.at[pl.ds(w * CH, CH)], idx_v)
    pltpu.async_copy(t_emb.at[idx_v], rows_v, gsem).wait()
    pltpu.sync_copy(rows_v, topic_rows.at[pl.ds(w * CH, CH)])

  plsc.subcore_barrier()

  _agg_groups(None, ww_dst, None, None, cww, None, didx_big, didx2, None,
              ones_v, gsem, ssem, s * WW_CPT, WW_CPT, 7, lo, WW_HALF)

  @pl.when(c == 0)
  def _():
    _agg_groups(None, wt_dst, None, None, ct, None, didx_big, didx2, None,
                ones_v, gsem, ssem, s * WT_CPT, WT_CPT, 8)

  @pl.when(c == 1)
  def _():
    _agg_groups(None, tt_dst, None, None, ct, None, didx_big, didx2, None,
                ones_v, gsem, ssem, s * TT_CPT, TT_CPT, 4)

  plsc.subcore_barrier()
  pltpu.sync_copy(cww.at[pl.ds(s * 504, 504)],
                  cnt_ww.at[pl.ds(lo + s * 504, 504)])

  @pl.when(c == 0)
  def _():
    pltpu.sync_copy(ct.at[pl.ds(s * 208, 208)],
                    cnt_wt.at[pl.ds(s * 208, 208)])

  @pl.when(c == 1)
  def _():
    pltpu.sync_copy(ct.at[pl.ds(s * 208, 208)],
                    cnt_tt.at[pl.ds(s * 208, 208)])


def _l1a_body(feat_w, src, dst, z128,
              s_ww,
              acc, sidx_big, didx_big, didx2, rows, sem_g, sem_s):
  c = lax.axis_index("c")
  s = lax.axis_index("s")
  lo = c * WW_HALF
  pltpu.sync_copy(z128.at[pl.ds(0, 512)], acc.at[pl.ds(s * 512, 512)])
  plsc.subcore_barrier()
  _agg_groups(src, dst, feat_w, acc, None, sidx_big, didx_big, didx2, rows,
              None, sem_g, sem_s, s * WW_CPT, WW_CPT, 3, lo, WW_HALF)
  plsc.subcore_barrier()
  pltpu.sync_copy(acc.at[pl.ds(s * 504, 504)],
                  s_ww.at[pl.ds(lo + s * 504, 504)])


def _l1b_body(feat_w, feat_t, wt_src, wt_dst, tt_src, tt_dst, z128,
              s_wt, s_tt,
              acc, sidx_big, didx_big, didx2, rows, sem_g, sem_s):
  c = lax.axis_index("c")
  s = lax.axis_index("s")
  pltpu.sync_copy(z128.at[pl.ds(0, 208)], acc.at[pl.ds(s * 208, 208)])
  plsc.subcore_barrier()

  @pl.when(c == 0)
  def _():
    _agg_groups(wt_src, wt_dst, feat_w, acc, None, sidx_big, didx_big,
                didx2, rows, None, sem_g, sem_s, s * WT_CPT, WT_CPT, 4)

  @pl.when(c == 1)
  def _():
    _agg_groups(tt_src, tt_dst, feat_t, acc, None, sidx_big, didx_big,
                didx2, rows, None, sem_g, sem_s, s * TT_CPT, TT_CPT, 4)

  plsc.subcore_barrier()

  @pl.when(c == 0)
  def _():
    pltpu.sync_copy(acc.at[pl.ds(s * 208, 208)],
                    s_wt.at[pl.ds(s * 208, 208)])

  @pl.when(c == 1)
  def _():
    pltpu.sync_copy(acc.at[pl.ds(s * 208, 208)],
                    s_tt.at[pl.ds(s * 208, 208)])


def _l2_body(feat_w, feat_t, wd_src, wd_dst, td_src, td_dst, z128, ones128,
             s_wd, s_td, cnt_wd, cnt_td,
             a_wd, a_td, cw, ct, sidx_big, didx_big, didx2, rows, ones_v,
             sem_g, sem_s):
  c = lax.axis_index("c")
  s = lax.axis_index("s")
  for a in (a_wd, a_td, cw, ct):
    pltpu.sync_copy(z128.at[pl.ds(0, 8)], a.at[pl.ds(s * 8, 8)])
  pltpu.sync_copy(ones128, ones_v)
  plsc.subcore_barrier()
  _agg_groups(wd_src, wd_dst, feat_w, a_wd, cw, sidx_big, didx_big, didx2,
              rows, ones_v, sem_g, sem_s, (c * NS + s) * WD_CPT, WD_CPT, 4)
  _agg_groups(td_src, td_dst, feat_t, a_td, ct, sidx_big, didx_big, didx2,
              rows, ones_v, sem_g, sem_s, (c * NS + s) * TD_CPT, TD_CPT, 2)
  plsc.subcore_barrier()
  for a, o in ((a_wd, s_wd), (a_td, s_td), (cw, cnt_wd), (ct, cnt_td)):
    pltpu.sync_copy(a.at[pl.ds(s * 8, 8)], o.at[c, pl.ds(s * 8, 8)])


def _adapt_body(rows_ref, w_ref, b_ref, o_ref):
  o_ref[...] = jnp.dot(rows_ref[...], w_ref[...],
                       preferred_element_type=F32) + b_ref[...]


def _word1_body(s_ref, c_ref, w_ref, b_ref, o_ref):
  cnt = c_ref[:, 0:1]
  num = jnp.dot(s_ref[...], w_ref[...],
                preferred_element_type=F32) + cnt * b_ref[...]
  o_ref[...] = jnp.maximum(num / jnp.maximum(cnt, 1.0), 0.0)


def _mean_block(sagg, cnt, w_ref, b_ref):
  num = jnp.dot(sagg, w_ref[...], preferred_element_type=F32) + cnt * b_ref[...]
  return num / jnp.maximum(cnt, 1.0)


def _topic1_body(swt_ref, cwt_ref, wwt_ref, bwt_ref,
                 stt_ref, ctt_ref, wtt_ref, btt_ref, o_ref):
  m = (_mean_block(swt_ref[...], cwt_ref[:, 0:1], wwt_ref, bwt_ref) +
       _mean_block(stt_ref[...], ctt_ref[:, 0:1], wtt_ref, btt_ref))
  o_ref[...] = jnp.maximum(m, 0.0)


def _head_body(swd_ref, cwd_ref, wwd_ref, bwd_ref,
               std_ref, ctd_ref, wtd_ref, btd_ref,
               ow_ref, ob_ref, y_ref, loss_ref, yp_ref):
  m = (_mean_block(swd_ref[0] + swd_ref[1],
                   cwd_ref[0][:, 0:1] + cwd_ref[1][:, 0:1],
                   wwd_ref, bwd_ref) +
       _mean_block(std_ref[0] + std_ref[1],
                   ctd_ref[0][:, 0:1] + ctd_ref[1][:, 0:1],
                   wtd_ref, btd_ref))
  feat = jnp.maximum(m[0:NDOC], 0.0)
  z = jnp.dot(feat, ow_ref[...], preferred_element_type=F32) + ob_ref[...]
  y = y_ref[...]
  loss = jnp.mean(jnp.maximum(z, 0.0) - z * y +
                  jnp.log1p(jnp.exp(-jnp.abs(z))))
  loss_ref[...] = loss[None, None]
  yp_ref[...] = jax.nn.sigmoid(z)


def kernel(word_ids, topic_ids, ww_src, ww_dst, wt_src, wt_dst, tt_src,
           tt_dst, wd_src, wd_dst, td_src, td_dst, doc_gid, y_data,
           W_word, topic_embeds, adapt_W, adapt_b, layers, out_W, out_b):
  mult = NW * CH
  ww_s, ww_d = _pad_edges(ww_src, ww_dst, NWORD, mult)
  wt_s, wt_d = _pad_edges(wt_src, wt_dst, NTOPIC, mult)
  tt_s, tt_d = _pad_edges(tt_src, tt_dst, NTOPIC, mult)
  wd_s, wd_d = _pad_edges(wd_src, wd_dst, NDOC, mult)
  td_s, td_d = _pad_edges(td_src, td_dst, NDOC, mult)
  word_ids = word_ids.astype(jnp.int32)
  topic_ids = topic_ids.astype(jnp.int32)

  z128 = jnp.zeros((512, 128), F32)
  ones128 = jnp.ones((CH, 128), F32)

  sds = jax.ShapeDtypeStruct

  prep = pl.kernel(
      _prep_body,
      out_type=(sds((NWORD, 128), F32), sds((NTOPIC, 128), F32),
                sds((NWORD_P, 128), F32), sds((NTOPIC_P, 128), F32),
                sds((NTOPIC_P, 128), F32)),
      mesh=_mesh(),
      scratch_types=(pltpu.VMEM_SHARED((WW_ACC, 128), F32),
                     pltpu.VMEM_SHARED((NTOPIC_P, 128), F32),
                     pltpu.VMEM((CH,), jnp.int32),
                     pltpu.VMEM((8 * CH,), jnp.int32),
                     pltpu.VMEM((8, CH), jnp.int32),
                     pltpu.VMEM((CH, 128), F32),
                     pltpu.VMEM((CH, 128), F32),
                     pltpu.SemaphoreType.DMA,
                     pltpu.SemaphoreType.DMA),
  )
  word_rows, feat_t0, cnt_ww, cnt_wt, cnt_tt = prep(
      W_word, word_ids, topic_embeds, topic_ids, ww_d, wt_d, tt_d,
      z128, ones128)

  feat_w0 = pl.pallas_call(
      _adapt_body,
      grid=(10,),
      in_specs=[pl.BlockSpec((1600, 128), lambda i: (i, 0)),
                pl.BlockSpec((128, 128), lambda i: (0, 0)),
                pl.BlockSpec((1, 128), lambda i: (0, 0))],
      out_specs=pl.BlockSpec((1600, 128), lambda i: (i, 0)),
      out_shape=sds((NWORD, 128), F32),
  )(word_rows, adapt_W, adapt_b.reshape(1, 128))

  l1a = pl.kernel(
      _l1a_body,
      out_type=sds((NWORD_P, 128), F32),
      mesh=_mesh(),
      scratch_types=(pltpu.VMEM_SHARED((WW_ACC, 128), F32),
                     pltpu.VMEM((3 * CH,), jnp.int32),
                     pltpu.VMEM((3 * CH,), jnp.int32),
                     pltpu.VMEM((3, CH), jnp.int32),
                     pltpu.VMEM((3, CH, 128), F32),
                     pltpu.SemaphoreType.DMA,
                     pltpu.SemaphoreType.DMA),
  )
  s_ww = l1a(feat_w0, ww_s, ww_d, z128)

  l1b = pl.kernel(
      _l1b_body,
      out_type=(sds((NTOPIC_P, 128), F32), sds((NTOPIC_P, 128), F32)),
      mesh=_mesh(),
      scratch_types=(pltpu.VMEM_SHARED((NTOPIC_P, 128), F32),
                     pltpu.VMEM((4 * CH,), jnp.int32),
                     pltpu.VMEM((4 * CH,), jnp.int32),
                     pltpu.VMEM((4, CH), jnp.int32),
                     pltpu.VMEM((4, CH, 128), F32),
                     pltpu.SemaphoreType.DMA,
                     pltpu.SemaphoreType.DMA),
  )
  s_wt, s_tt = l1b(feat_w0, feat_t0, wt_s, wt_d, tt_s, tt_d, z128)

  lp1, lp2 = layers[0], layers[1]

  feat_w1 = pl.pallas_call(
      _word1_body,
      grid=(16,),
      in_specs=[pl.BlockSpec((1008, 128), lambda i: (i, 0)),
                pl.BlockSpec((1008, 128), lambda i: (i, 0)),
                pl.BlockSpec((128, 128), lambda i: (0, 0)),
                pl.BlockSpec((1, 128), lambda i: (0, 0))],
      out_specs=pl.BlockSpec((1008, 128), lambda i: (i, 0)),
      out_shape=sds((NWORD_P, 128), F32),
  )(s_ww, cnt_ww, lp1['ww']['W'], lp1['ww']['b'].reshape(1, 128))

  feat_t1 = pl.pallas_call(
      _topic1_body,
      in_specs=[pl.BlockSpec((NTOPIC_P, 128), lambda: (0, 0)),
                pl.BlockSpec((NTOPIC_P, 128), lambda: (0, 0)),
                pl.BlockSpec((128, 128), lambda: (0, 0)),
                pl.BlockSpec((1, 128), lambda: (0, 0)),
                pl.BlockSpec((NTOPIC_P, 128), lambda: (0, 0)),
                pl.BlockSpec((NTOPIC_P, 128), lambda: (0, 0)),
                pl.BlockSpec((128, 128), lambda: (0, 0)),
                pl.BlockSpec((1, 128), lambda: (0, 0))],
      out_specs=pl.BlockSpec((NTOPIC_P, 128), lambda: (0, 0)),
      out_shape=sds((NTOPIC_P, 128), F32),
  )(s_wt, cnt_wt, lp1['wt']['W'], lp1['wt']['b'].reshape(1, 128),
    s_tt, cnt_tt, lp1['tt']['W'], lp1['tt']['b'].reshape(1, 128))

  l2 = pl.kernel(
      _l2_body,
      out_type=(sds((NC, NDOC_P, 128), F32), sds((NC, NDOC_P, 128), F32),
                sds((NC, NDOC_P, 128), F32), sds((NC, NDOC_P, 128), F32)),
      mesh=_mesh(),
      scratch_types=(pltpu.VMEM_SHARED((NDOC_P, 128), F32),
                     pltpu.VMEM_SHARED((NDOC_P, 128), F32),
                     pltpu.VMEM_SHARED((NDOC_P, 128), F32),
                     pltpu.VMEM_SHARED((NDOC_P, 128), F32),
                     pltpu.VMEM((4 * CH,), jnp.int32),
                     pltpu.VMEM((4 * CH,), jnp.int32),
                     pltpu.VMEM((4, CH), jnp.int32),
                     pltpu.VMEM((4, CH, 128), F32),
                     pltpu.VMEM((CH, 128), F32),
                     pltpu.SemaphoreType.DMA,
                     pltpu.SemaphoreType.DMA),
  )
  s_wd, s_td, cnt_wd, cnt_td = l2(feat_w1, feat_t1, wd_s, wd_d, td_s, td_d,
                                  z128, ones128)

  loss2d, yp2d = pl.pallas_call(
      _head_body,
      in_specs=[pl.BlockSpec((NC, NDOC_P, 128), lambda: (0, 0, 0)),
                pl.BlockSpec((NC, NDOC_P, 128), lambda: (0, 0, 0)),
                pl.BlockSpec((128, 128), lambda: (0, 0)),
                pl.BlockSpec((1, 128), lambda: (0, 0)),
                pl.BlockSpec((NC, NDOC_P, 128), lambda: (0, 0, 0)),
                pl.BlockSpec((NC, NDOC_P, 128), lambda: (0, 0, 0)),
                pl.BlockSpec((128, 128), lambda: (0, 0)),
                pl.BlockSpec((1, 128), lambda: (0, 0)),
                pl.BlockSpec((128, 1), lambda: (0, 0)),
                pl.BlockSpec((1, 1), lambda: (0, 0)),
                pl.BlockSpec((NDOC, 1), lambda: (0, 0))],
      out_specs=[pl.BlockSpec((1, 1), lambda: (0, 0)),
                 pl.BlockSpec((NDOC, 1), lambda: (0, 0))],
      out_shape=(sds((1, 1), F32), sds((NDOC, 1), F32)),
  )(s_wd, cnt_wd, lp2['wd']['W'], lp2['wd']['b'].reshape(1, 128),
    s_td, cnt_td, lp2['td']['W'], lp2['td']['b'].reshape(1, 128),
    out_W, out_b.reshape(1, 1), y_data.reshape(NDOC, 1))

  return (loss2d.reshape(()), yp2d.reshape(NDOC))


# speedup vs baseline: 3.0969x; 1.0549x over previous
import functools
import jax
import jax.numpy as jnp
from jax import lax
from jax.experimental import pallas as pl
from jax.experimental.pallas import tpu as pltpu
from jax.experimental.pallas import tpu_sc as plsc

NC, NS = 2, 16
NW = NC * NS
CH = 128

NWORD, NWORD_P = 16000, 16128
NTOPIC, NTOPIC_P = 3200, 3328
NDOC, NDOC_P = 64, 128

WW_HALF = 8064
WW_ACC = 8192

WW_E, WW_CPT = 258048, 126
WT_E, WT_CPT, WT_CNT_CPT = 65536, 16, 32
TT_E, TT_CPT, TT_CNT_CPT = 8192, 2, 4
WD_E, WD_CPT = 131072, 32
TD_E, TD_CPT = 8192, 2

F32 = jnp.float32
_mesh = functools.partial(plsc.VectorSubcoreMesh,
                          core_axis_name="c", subcore_axis_name="s")


def _pad_edges(src, dst, sac_row, mult):
  e = src.shape[0]
  ep = ((e + mult - 1) // mult) * mult
  src = src.astype(jnp.int32)
  dst = dst.astype(jnp.int32)
  if ep != e:
    src = jnp.concatenate([src, jnp.zeros((ep - e,), jnp.int32)])
    dst = jnp.concatenate([dst, jnp.full((ep - e,), sac_row, jnp.int32)])
  return src, dst


def _remap(didx_big, p, b, didx2, lo, n):
  for k in range(CH // 16):
    d = didx_big[p, pl.ds(b * CH + k * 16, 16)]
    if lo is not None:
      d = d - lo
      d = jnp.where((d >= 0) & (d < n), d, n)
    didx2[b, pl.ds(k * 16, 16)] = d


def _agg_groups(esrc, edst, feat, acc, cacc, sidx_big, didx_big, didx2,
                rows, ones_v, gsem, ssem, isem, base_chunk, n_chunks, K,
                lo=None, nrange=None):
  ng = n_chunks // K

  def fire_idx(g, p):
    off0 = (base_chunk + g * K) * CH
    pltpu.async_copy(edst.at[pl.ds(off0, K * CH)],
                     didx_big.at[p, pl.ds(0, K * CH)], isem)
    if feat is not None:
      pltpu.async_copy(esrc.at[pl.ds(off0, K * CH)],
                       sidx_big.at[p, pl.ds(0, K * CH)], isem)

  def drain_idx(g, p):
    off0 = (base_chunk + g * K) * CH
    pltpu.make_async_copy(edst.at[pl.ds(off0, K * CH)],
                          didx_big.at[p, pl.ds(0, K * CH)], isem).wait()
    if feat is not None:
      pltpu.make_async_copy(esrc.at[pl.ds(off0, K * CH)],
                            sidx_big.at[p, pl.ds(0, K * CH)], isem).wait()

  def work(g, p):
    if feat is not None:
      for b in range(K):
        pltpu.async_copy(feat.at[sidx_big.at[p, pl.ds(b * CH, CH)]],
                         rows.at[b], gsem)

    @pl.when(g + 1 < ng)
    def _():
      fire_idx(g + 1, 1 - p)

    for b in range(K):
      _remap(didx_big, p, b, didx2, lo, nrange)
    if cacc is not None:
      for b in range(K):
        pltpu.async_copy(ones_v, cacc.at[didx2.at[b]], ssem, add=True)
    if feat is not None:
      for b in range(K):
        pltpu.make_async_copy(feat.at[sidx_big.at[p, pl.ds(b * CH, CH)]],
                              rows.at[b], gsem).wait()
      for b in range(K):
        pltpu.async_copy(rows.at[b], acc.at[didx2.at[b]], ssem, add=True)
      for b in range(K):
        pltpu.make_async_copy(rows.at[b], acc.at[didx2.at[b]], ssem).wait()
    if cacc is not None:
      for b in range(K):
        pltpu.make_async_copy(ones_v, cacc.at[didx2.at[b]], ssem).wait()

  fire_idx(0, 0)

  def grp2(h, _):
    g = h * 2
    drain_idx(g, 0)
    work(g, 0)

    @pl.when(g + 1 < ng)
    def _():
      drain_idx(g + 1, 1)
      work(g + 1, 1)
    return 0

  lax.fori_loop(0, (ng + 1) // 2, grp2, 0)


def _prep_body(w_word, word_ids, t_emb, topic_ids, ww_dst, wt_dst, tt_dst,
               z128, ones128,
               word_rows, topic_rows, cnt_ww, cnt_wt, cnt_tt,
               cww, ct, idx_v, didx_big, didx2, rows_v, ones_v, gsem, ssem, isem):
  c = lax.axis_index("c")
  s = lax.axis_index("s")
  w = s * NC + c
  lo = c * WW_HALF
  pltpu.sync_copy(z128.at[pl.ds(0, 512)], cww.at[pl.ds(s * 512, 512)])
  pltpu.sync_copy(z128.at[pl.ds(0, 208)], ct.at[pl.ds(s * 208, 208)])
  pltpu.sync_copy(ones128, ones_v)

  def wg(i, _):
    g = w + i * NW

    @pl.when(g < 125)
    def _():
      pltpu.sync_copy(word_ids.at[pl.ds(g * CH, CH)], idx_v)
      pltpu.async_copy(w_word.at[idx_v], rows_v, gsem).wait()
      pltpu.sync_copy(rows_v, word_rows.at[pl.ds(g * CH, CH)])
    return 0

  lax.fori_loop(0, 4, wg, 0)

  @pl.when(w < 25)
  def _():
    pltpu.sync_copy(topic_ids.at[pl.ds(w * CH, CH)], idx_v)
    pltpu.async_copy(t_emb.at[idx_v], rows_v, gsem).wait()
    pltpu.sync_copy(rows_v, topic_rows.at[pl.ds(w * CH, CH)])

  plsc.subcore_barrier()

  _agg_groups(None, ww_dst, None, None, cww, None, didx_big, didx2, None,
              ones_v, gsem, ssem, isem, s * WW_CPT, WW_CPT, 7, lo, WW_HALF)

  @pl.when(c == 0)
  def _():
    _agg_groups(None, wt_dst, None, None, ct, None, didx_big, didx2, None,
                ones_v, gsem, ssem, isem, s * WT_CNT_CPT, WT_CNT_CPT, 8)

  @pl.when(c == 1)
  def _():
    _agg_groups(None, tt_dst, None, None, ct, None, didx_big, didx2, None,
                ones_v, gsem, ssem, isem, s * TT_CNT_CPT, TT_CNT_CPT, 4)

  plsc.subcore_barrier()
  pltpu.sync_copy(cww.at[pl.ds(s * 504, 504)],
                  cnt_ww.at[pl.ds(lo + s * 504, 504)])

  @pl.when(c == 0)
  def _():
    pltpu.sync_copy(ct.at[pl.ds(s * 208, 208)],
                    cnt_wt.at[pl.ds(s * 208, 208)])

  @pl.when(c == 1)
  def _():
    pltpu.sync_copy(ct.at[pl.ds(s * 208, 208)],
                    cnt_tt.at[pl.ds(s * 208, 208)])


def _l1_body(feat_w, feat_t, ww_src, ww_dst, wt_src, wt_dst, tt_src,
             tt_dst, z128,
             s_ww, s_wt, s_tt,
             acc, ct, sidx_big, didx_big, didx2, rows, sem_g, sem_s, sem_i):
  c = lax.axis_index("c")
  s = lax.axis_index("s")
  lo = c * WW_HALF
  pltpu.sync_copy(z128.at[pl.ds(0, 512)], acc.at[pl.ds(s * 512, 512)])
  pltpu.sync_copy(z128.at[pl.ds(0, 208)], ct.at[pl.ds(s * 208, 208)])
  plsc.subcore_barrier()
  _agg_groups(ww_src, ww_dst, feat_w, acc, None, sidx_big, didx_big, didx2,
              rows, None, sem_g, sem_s, sem_i, s * WW_CPT, WW_CPT, 2,
              lo, WW_HALF)

  @pl.when(c == 0)
  def _():
    _agg_groups(wt_src, wt_dst, feat_w, ct, None, sidx_big, didx_big, didx2,
                rows, None, sem_g, sem_s, sem_i, s * WT_CNT_CPT,
                WT_CNT_CPT, 2)

  @pl.when(c == 1)
  def _():
    _agg_groups(tt_src, tt_dst, feat_t, ct, None, sidx_big, didx_big, didx2,
                rows, None, sem_g, sem_s, sem_i, s * TT_CNT_CPT,
                TT_CNT_CPT, 2)

  plsc.subcore_barrier()
  pltpu.sync_copy(acc.at[pl.ds(s * 504, 504)],
                  s_ww.at[pl.ds(lo + s * 504, 504)])

  @pl.when(c == 0)
  def _():
    pltpu.sync_copy(ct.at[pl.ds(s * 208, 208)],
                    s_wt.at[pl.ds(s * 208, 208)])

  @pl.when(c == 1)
  def _():
    pltpu.sync_copy(ct.at[pl.ds(s * 208, 208)],
                    s_tt.at[pl.ds(s * 208, 208)])


def _l2_body(feat_w, feat_t, wd_src, wd_dst, td_src, td_dst, z128, ones128,
             s_wd, s_td, cnt_wd, cnt_td,
             acc, cacc, red_v, out_v, sidx_big, didx_big, didx2, rows,
             ones_v, sem_g, sem_s, sem_i):
  c = lax.axis_index("c")
  s = lax.axis_index("s")
  sbase = s * NDOC_P
  pltpu.sync_copy(ones128, ones_v)

  def run(esrc, edst, feat, n_cpt, k, s_out, c_out):
    pltpu.sync_copy(z128.at[pl.ds(0, NDOC_P)], acc.at[pl.ds(sbase, NDOC_P)])
    pltpu.sync_copy(z128.at[pl.ds(0, NDOC_P)], cacc.at[pl.ds(sbase, NDOC_P)])
    _agg_groups(esrc, edst, feat, acc, cacc, sidx_big, didx_big, didx2,
                rows, ones_v, sem_g, sem_s, sem_i, (c * NS + s) * n_cpt,
                n_cpt, k, -sbase, NS * NDOC_P)
    plsc.subcore_barrier()
    for a, o in ((acc, s_out), (cacc, c_out)):
      for h in range(2):
        for r in range(8):
          pltpu.sync_copy(a.at[pl.ds((h * 8 + r) * NDOC_P + s * 8, 8)],
                          red_v.at[r])

        def red(jq, _, h=h):
          j = jq // 8
          q = (jq % 8) * 16
          t = red_v[0, j, pl.ds(q, 16)]
          for r in range(1, 8):
            t = t + red_v[r, j, pl.ds(q, 16)]
          if h == 1:
            t = t + out_v[j, pl.ds(q, 16)]
          out_v[j, pl.ds(q, 16)] = t
          return 0

        lax.fori_loop(0, 64, red, 0)
      pltpu.sync_copy(out_v, o.at[c, pl.ds(s * 8, 8)])
    plsc.subcore_barrier()

  run(wd_src, wd_dst, feat_w, WD_CPT, 4, s_wd, cnt_wd)
  run(td_src, td_dst, feat_t, TD_CPT, 2, s_td, cnt_td)


def _adapt_body(rows_ref, w_ref, b_ref, o_ref):
  o_ref[...] = jnp.dot(rows_ref[...], w_ref[...],
                       preferred_element_type=F32) + b_ref[...]


def _word1_body(s_ref, c_ref, w_ref, b_ref, o_ref):
  cnt = c_ref[:, 0:1]
  num = jnp.dot(s_ref[...], w_ref[...],
                preferred_element_type=F32) + cnt * b_ref[...]
  o_ref[...] = jnp.maximum(num / jnp.maximum(cnt, 1.0), 0.0)


def _mean_block(sagg, cnt, w_ref, b_ref):
  num = jnp.dot(sagg, w_ref[...], preferred_element_type=F32) + cnt * b_ref[...]
  return num / jnp.maximum(cnt, 1.0)


def _topic1_body(swt_ref, cwt_ref, wwt_ref, bwt_ref,
                 stt_ref, ctt_ref, wtt_ref, btt_ref, o_ref):
  m = (_mean_block(swt_ref[...], cwt_ref[:, 0:1], wwt_ref, bwt_ref) +
       _mean_block(stt_ref[...], ctt_ref[:, 0:1], wtt_ref, btt_ref))
  o_ref[...] = jnp.maximum(m, 0.0)


def _head_body(swd_ref, cwd_ref, wwd_ref, bwd_ref,
               std_ref, ctd_ref, wtd_ref, btd_ref,
               ow_ref, ob_ref, y_ref, loss_ref, yp_ref):
  m = (_mean_block(swd_ref[0] + swd_ref[1],
                   cwd_ref[0][:, 0:1] + cwd_ref[1][:, 0:1],
                   wwd_ref, bwd_ref) +
       _mean_block(std_ref[0] + std_ref[1],
                   ctd_ref[0][:, 0:1] + ctd_ref[1][:, 0:1],
                   wtd_ref, btd_ref))
  feat = jnp.maximum(m[0:NDOC], 0.0)
  z = jnp.dot(feat, ow_ref[...], preferred_element_type=F32) + ob_ref[...]
  y = y_ref[...]
  loss = jnp.mean(jnp.maximum(z, 0.0) - z * y +
                  jnp.log1p(jnp.exp(-jnp.abs(z))))
  loss_ref[...] = loss[None, None]
  yp_ref[...] = jax.nn.sigmoid(z)


def kernel(word_ids, topic_ids, ww_src, ww_dst, wt_src, wt_dst, tt_src,
           tt_dst, wd_src, wd_dst, td_src, td_dst, doc_gid, y_data,
           W_word, topic_embeds, adapt_W, adapt_b, layers, out_W, out_b):
  mult = NW * CH
  ww_s, ww_d = _pad_edges(ww_src, ww_dst, NWORD, mult)
  wt_s, wt_d = _pad_edges(wt_src, wt_dst, NTOPIC, mult)
  tt_s, tt_d = _pad_edges(tt_src, tt_dst, NTOPIC, mult)
  wd_s, wd_d = _pad_edges(wd_src, wd_dst, NDOC, mult)
  td_s, td_d = _pad_edges(td_src, td_dst, NDOC, mult)
  word_ids = word_ids.astype(jnp.int32)
  topic_ids = topic_ids.astype(jnp.int32)

  z128 = jnp.zeros((512, 128), F32)
  ones128 = jnp.ones((CH, 128), F32)

  sds = jax.ShapeDtypeStruct

  prep = pl.kernel(
      _prep_body,
      out_type=(sds((NWORD, 128), F32), sds((NTOPIC, 128), F32),
                sds((NWORD_P, 128), F32), sds((NTOPIC_P, 128), F32),
                sds((NTOPIC_P, 128), F32)),
      mesh=_mesh(),
      scratch_types=(pltpu.VMEM_SHARED((WW_ACC, 128), F32),
                     pltpu.VMEM_SHARED((NTOPIC_P, 128), F32),
                     pltpu.VMEM((CH,), jnp.int32),
                     pltpu.VMEM((2, 8 * CH), jnp.int32),
                     pltpu.VMEM((8, CH), jnp.int32),
                     pltpu.VMEM((CH, 128), F32),
                     pltpu.VMEM((CH, 128), F32),
                     pltpu.SemaphoreType.DMA,
                     pltpu.SemaphoreType.DMA,
                     pltpu.SemaphoreType.DMA),
  )
  word_rows, feat_t0, cnt_ww, cnt_wt, cnt_tt = prep(
      W_word, word_ids, topic_embeds, topic_ids, ww_d, wt_d, tt_d,
      z128, ones128)

  feat_w0 = pl.pallas_call(
      _adapt_body,
      grid=(10,),
      in_specs=[pl.BlockSpec((1600, 128), lambda i: (i, 0)),
                pl.BlockSpec((128, 128), lambda i: (0, 0)),
                pl.BlockSpec((1, 128), lambda i: (0, 0))],
      out_specs=pl.BlockSpec((1600, 128), lambda i: (i, 0)),
      out_shape=sds((NWORD, 128), F32),
  )(word_rows, adapt_W, adapt_b.reshape(1, 128))

  l1 = pl.kernel(
      _l1_body,
      out_type=(sds((NWORD_P, 128), F32), sds((NTOPIC_P, 128), F32),
                sds((NTOPIC_P, 128), F32)),
      mesh=_mesh(),
      scratch_types=(pltpu.VMEM_SHARED((WW_ACC, 128), F32),
                     pltpu.VMEM_SHARED((NTOPIC_P, 128), F32),
                     pltpu.VMEM((2, 2 * CH), jnp.int32),
                     pltpu.VMEM((2, 2 * CH), jnp.int32),
                     pltpu.VMEM((2, CH), jnp.int32),
                     pltpu.VMEM((2, CH, 128), F32),
                     pltpu.SemaphoreType.DMA,
                     pltpu.SemaphoreType.DMA,
                     pltpu.SemaphoreType.DMA),
  )
  s_ww, s_wt, s_tt = l1(feat_w0, feat_t0, ww_s, ww_d, wt_s, wt_d,
                        tt_s, tt_d, z128)

  lp1, lp2 = layers[0], layers[1]

  feat_w1 = pl.pallas_call(
      _word1_body,
      grid=(16,),
      in_specs=[pl.BlockSpec((1008, 128), lambda i: (i, 0)),
                pl.BlockSpec((1008, 128), lambda i: (i, 0)),
                pl.BlockSpec((128, 128), lambda i: (0, 0)),
                pl.BlockSpec((1, 128), lambda i: (0, 0))],
      out_specs=pl.BlockSpec((1008, 128), lambda i: (i, 0)),
      out_shape=sds((NWORD_P, 128), F32),
  )(s_ww, cnt_ww, lp1['ww']['W'], lp1['ww']['b'].reshape(1, 128))

  feat_t1 = pl.pallas_call(
      _topic1_body,
      in_specs=[pl.BlockSpec((NTOPIC_P, 128), lambda: (0, 0)),
                pl.BlockSpec((NTOPIC_P, 128), lambda: (0, 0)),
                pl.BlockSpec((128, 128), lambda: (0, 0)),
                pl.BlockSpec((1, 128), lambda: (0, 0)),
                pl.BlockSpec((NTOPIC_P, 128), lambda: (0, 0)),
                pl.BlockSpec((NTOPIC_P, 128), lambda: (0, 0)),
                pl.BlockSpec((128, 128), lambda: (0, 0)),
                pl.BlockSpec((1, 128), lambda: (0, 0))],
      out_specs=pl.BlockSpec((NTOPIC_P, 128), lambda: (0, 0)),
      out_shape=sds((NTOPIC_P, 128), F32),
  )(s_wt, cnt_wt, lp1['wt']['W'], lp1['wt']['b'].reshape(1, 128),
    s_tt, cnt_tt, lp1['tt']['W'], lp1['tt']['b'].reshape(1, 128))

  l2 = pl.kernel(
      _l2_body,
      out_type=(sds((NC, NDOC_P, 128), F32), sds((NC, NDOC_P, 128), F32),
                sds((NC, NDOC_P, 128), F32), sds((NC, NDOC_P, 128), F32)),
      mesh=_mesh(),
      scratch_types=(pltpu.VMEM_SHARED((NS * NDOC_P, 128), F32),
                     pltpu.VMEM_SHARED((NS * NDOC_P, 128), F32),
                     pltpu.VMEM((8, 8, 128), F32),
                     pltpu.VMEM((8, 128), F32),
                     pltpu.VMEM((2, 4 * CH), jnp.int32),
                     pltpu.VMEM((2, 4 * CH), jnp.int32),
                     pltpu.VMEM((4, CH), jnp.int32),
                     pltpu.VMEM((4, CH, 128), F32),
                     pltpu.VMEM((CH, 128), F32),
                     pltpu.SemaphoreType.DMA,
                     pltpu.SemaphoreType.DMA,
                     pltpu.SemaphoreType.DMA),
  )
  s_wd, s_td, cnt_wd, cnt_td = l2(feat_w1, feat_t1, wd_s, wd_d, td_s, td_d,
                                  z128, ones128)

  loss2d, yp2d = pl.pallas_call(
      _head_body,
      in_specs=[pl.BlockSpec((NC, NDOC_P, 128), lambda: (0, 0, 0)),
                pl.BlockSpec((NC, NDOC_P, 128), lambda: (0, 0, 0)),
                pl.BlockSpec((128, 128), lambda: (0, 0)),
                pl.BlockSpec((1, 128), lambda: (0, 0)),
                pl.BlockSpec((NC, NDOC_P, 128), lambda: (0, 0, 0)),
                pl.BlockSpec((NC, NDOC_P, 128), lambda: (0, 0, 0)),
                pl.BlockSpec((128, 128), lambda: (0, 0)),
                pl.BlockSpec((1, 128), lambda: (0, 0)),
                pl.BlockSpec((128, 1), lambda: (0, 0)),
                pl.BlockSpec((1, 1), lambda: (0, 0)),
                pl.BlockSpec((NDOC, 1), lambda: (0, 0))],
      out_specs=[pl.BlockSpec((1, 1), lambda: (0, 0)),
                 pl.BlockSpec((NDOC, 1), lambda: (0, 0))],
      out_shape=(sds((1, 1), F32), sds((NDOC, 1), F32)),
  )(s_wd, cnt_wd, lp2['wd']['W'], lp2['wd']['b'].reshape(1, 128),
    s_td, cnt_td, lp2['td']['W'], lp2['td']['b'].reshape(1, 128),
    out_W, out_b.reshape(1, 1), y_data.reshape(NDOC, 1))

  return (loss2d.reshape(()), yp2d.reshape(NDOC))
